# baseline (speedup 1.0000x reference)
import functools

import jax
import jax.numpy as jnp
from jax import lax
from jax.experimental import pallas as pl
from jax.experimental.pallas import tpu as pltpu
from jax.experimental.pallas import tpu_sc as plsc

VOCAB = 1000000
HIDDEN = 128
MAX_LEN = 2048
BATCH = 4
NSEG = 3

NUM_CORES = 2
NUM_SUBCORES = 16
NW = NUM_CORES * NUM_SUBCORES
ROWS = BATCH * MAX_LEN
R_PER_W = ROWS // NW
CH = 128
NCH = R_PER_W // CH
LBLK = 256

_mesh = plsc.VectorSubcoreMesh(core_axis_name="c", subcore_axis_name="s")


def _comb_body(segtab_ref, pe_ref, out_ref):
    pe = pe_ref[...]
    for s in range(NSEG):
        out_ref[s] = pe + segtab_ref[s, :][None, :]


@jax.jit
def _comb_table(segment_table, pe):
    return pl.pallas_call(
        _comb_body,
        grid=(MAX_LEN // LBLK,),
        in_specs=[
            pl.BlockSpec((NSEG, HIDDEN), lambda i: (0, 0)),
            pl.BlockSpec((LBLK, HIDDEN), lambda i: (i, 0)),
        ],
        out_specs=pl.BlockSpec((NSEG, LBLK, HIDDEN), lambda i: (0, i, 0)),
        out_shape=jax.ShapeDtypeStruct((NSEG, MAX_LEN, HIDDEN), jnp.float32),
    )(segment_table, pe)


@functools.partial(
    pl.kernel,
    mesh=_mesh,
    out_type=jax.ShapeDtypeStruct((ROWS, HIDDEN), jnp.float32),
    scratch_types=[
        pltpu.VMEM((NCH, CH), jnp.int32),
        pltpu.VMEM((NCH, CH), jnp.int32),
        pltpu.VMEM((R_PER_W, HIDDEN), jnp.float32),
        pltpu.SemaphoreType.DMA,
        [pltpu.SemaphoreType.DMA] * NCH,
        pltpu.SemaphoreType.DMA,
    ],
)
def _embed_sc(tok_hbm, comb_hbm, x_hbm, combidx_hbm, out_hbm,
              tok_idx, comb_idx, acc, sem, gsems, osem):
    wid = lax.axis_index("s") * NUM_CORES + lax.axis_index("c")
    base = wid * R_PER_W
    b = wid // (MAX_LEN // R_PER_W)
    l0 = base % MAX_LEN

    hs = []
    for j in range(NCH):
        src = pl.ds(l0 + j * CH, CH)
        hs.append(pltpu.async_copy(x_hbm.at[b, src], tok_idx.at[j], sem))
        hs.append(pltpu.async_copy(combidx_hbm.at[b, src], comb_idx.at[j],
                                   sem))
    for h in hs:
        h.wait()

    combs = []
    for j in range(NCH):
        dst = acc.at[pl.ds(j * CH, CH)]
        combs.append(
            pltpu.async_copy(comb_hbm.at[comb_idx.at[j]], dst, gsems[j]))
    toks = []
    for j in range(NCH):
        combs[j].wait()
        dst = acc.at[pl.ds(j * CH, CH)]
        toks.append(
            pltpu.async_copy(tok_hbm.at[tok_idx.at[j]], dst, gsems[j],
                             add=True))
    outs = []
    for j in range(NCH):
        toks[j].wait()
        outs.append(
            pltpu.async_copy(acc.at[pl.ds(j * CH, CH)],
                             out_hbm.at[pl.ds(base + j * CH, CH)], osem))
    for h in outs:
        h.wait()


@jax.jit
def kernel(x, segment, token_table, segment_table, pe):
    comb = _comb_table(segment_table, pe).reshape(NSEG * MAX_LEN, HIDDEN)
    comb_idx = segment * MAX_LEN + jnp.arange(MAX_LEN, dtype=jnp.int32)[None, :]
    out = _embed_sc(token_table, comb, x, comb_idx)
    return out.reshape(BATCH, MAX_LEN, HIDDEN)


# speedup vs baseline: 1.0933x; 1.0933x over previous
import functools

import jax
import jax.numpy as jnp
from jax import lax
from jax.experimental import pallas as pl
from jax.experimental.pallas import tpu as pltpu
from jax.experimental.pallas import tpu_sc as plsc

VOCAB = 1000000
HIDDEN = 128
MAX_LEN = 2048
BATCH = 4
NSEG = 3

NUM_CORES = 2
NUM_SUBCORES = 16
NW = NUM_CORES * NUM_SUBCORES
ROWS = BATCH * MAX_LEN
R_PER_W = ROWS // NW
CH = 128
NCH = R_PER_W // CH
LBLK = 256

_mesh = plsc.VectorSubcoreMesh(core_axis_name="c", subcore_axis_name="s")


def _comb_body(segtab_ref, pe_ref, out_ref):
    pe = pe_ref[...]
    for s in range(NSEG):
        out_ref[s] = pe + segtab_ref[s, :][None, :]


@jax.jit
def _comb_table(segment_table, pe):
    return pl.pallas_call(
        _comb_body,
        out_shape=jax.ShapeDtypeStruct((NSEG, MAX_LEN, HIDDEN), jnp.float32),
    )(segment_table, pe)


@functools.partial(
    pl.kernel,
    mesh=_mesh,
    out_type=jax.ShapeDtypeStruct((ROWS, HIDDEN), jnp.float32),
    scratch_types=[
        pltpu.VMEM((NCH, CH), jnp.int32),
        pltpu.VMEM((NCH, CH), jnp.int32),
        pltpu.VMEM((R_PER_W, HIDDEN), jnp.float32),
        pltpu.SemaphoreType.DMA,
        [pltpu.SemaphoreType.DMA] * NCH,
        pltpu.SemaphoreType.DMA,
    ],
)
def _embed_sc(tok_hbm, comb_hbm, x_hbm, combidx_hbm, out_hbm,
              tok_idx, comb_idx, acc, sem, gsems, osem):
    wid = lax.axis_index("s") * NUM_CORES + lax.axis_index("c")
    base = wid * R_PER_W
    b = wid // (MAX_LEN // R_PER_W)
    l0 = base % MAX_LEN

    hs = []
    for j in range(NCH):
        src = pl.ds(l0 + j * CH, CH)
        hs.append(pltpu.async_copy(x_hbm.at[b, src], tok_idx.at[j], sem))
        hs.append(pltpu.async_copy(combidx_hbm.at[b, src], comb_idx.at[j],
                                   sem))
    for h in hs:
        h.wait()

    combs = []
    for j in range(NCH):
        dst = acc.at[pl.ds(j * CH, CH)]
        combs.append(
            pltpu.async_copy(comb_hbm.at[comb_idx.at[j]], dst, gsems[j]))
    toks = []
    for j in range(NCH):
        combs[j].wait()
        dst = acc.at[pl.ds(j * CH, CH)]
        toks.append(
            pltpu.async_copy(tok_hbm.at[tok_idx.at[j]], dst, gsems[j],
                             add=True))
    outs = []
    for j in range(NCH):
        toks[j].wait()
        outs.append(
            pltpu.async_copy(acc.at[pl.ds(j * CH, CH)],
                             out_hbm.at[pl.ds(base + j * CH, CH)], osem))
    for h in outs:
        h.wait()


@jax.jit
def kernel(x, segment, token_table, segment_table, pe):
    comb = _comb_table(segment_table, pe).reshape(NSEG * MAX_LEN, HIDDEN)
    comb_idx = segment * MAX_LEN + jnp.arange(MAX_LEN, dtype=jnp.int32)[None, :]
    out = _embed_sc(token_table, comb, x, comb_idx)
    return out.reshape(BATCH, MAX_LEN, HIDDEN)


# speedup vs baseline: 1.1479x; 1.0499x over previous
import functools

import jax
import jax.numpy as jnp
from jax import lax
from jax.experimental import pallas as pl
from jax.experimental.pallas import tpu as pltpu
from jax.experimental.pallas import tpu_sc as plsc

VOCAB = 1000000
HIDDEN = 128
MAX_LEN = 2048
BATCH = 4
NSEG = 3

NUM_CORES = 2
NUM_SUBCORES = 16
NW = NUM_CORES * NUM_SUBCORES
ROWS = BATCH * MAX_LEN
R_PER_W = ROWS // NW
CH = 128
NCH = R_PER_W // CH
LANES = 16

_mesh = plsc.VectorSubcoreMesh(core_axis_name="c", subcore_axis_name="s")


def _comb_body(segtab_ref, pe_ref, out_ref):
    pe = pe_ref[...]
    for s in range(NSEG):
        out_ref[s] = pe + segtab_ref[s, :][None, :]


@jax.jit
def _comb_table(segment_table, pe):
    return pl.pallas_call(
        _comb_body,
        out_shape=jax.ShapeDtypeStruct((NSEG, MAX_LEN, HIDDEN), jnp.float32),
    )(segment_table, pe)


@functools.partial(
    pl.kernel,
    mesh=_mesh,
    out_type=jax.ShapeDtypeStruct((ROWS, HIDDEN), jnp.float32),
    scratch_types=[
        pltpu.VMEM((NCH, CH), jnp.int32),
        pltpu.VMEM((NCH, CH), jnp.int32),
        pltpu.VMEM((R_PER_W, HIDDEN), jnp.float32),
        pltpu.SemaphoreType.DMA,
        [pltpu.SemaphoreType.DMA] * NCH,
        pltpu.SemaphoreType.DMA,
    ],
)
def _embed_sc(tok_hbm, comb_hbm, x_hbm, seg_hbm, out_hbm,
              tok_idx, comb_idx, acc, sem, gsems, osem):
    wid = lax.axis_index("s") * NUM_CORES + lax.axis_index("c")
    base = wid * R_PER_W
    b = wid // (MAX_LEN // R_PER_W)
    l0 = base % MAX_LEN

    ht = [pltpu.async_copy(x_hbm.at[b, pl.ds(l0 + j * CH, CH)],
                           tok_idx.at[j], sem) for j in range(NCH)]
    hseg = [pltpu.async_copy(seg_hbm.at[b, pl.ds(l0 + j * CH, CH)],
                             comb_idx.at[j], sem) for j in range(NCH)]
    toks = []
    for j in range(NCH):
        ht[j].wait()
        toks.append(
            pltpu.async_copy(tok_hbm.at[tok_idx.at[j]],
                             acc.at[pl.ds(j * CH, CH)], gsems[j]))

    for h in hseg:
        h.wait()
    iota = lax.iota(jnp.int32, LANES)
    for j in range(NCH):
        for c in range(CH // LANES):
            s = comb_idx[j, pl.ds(c * LANES, LANES)]
            comb_idx[j, pl.ds(c * LANES, LANES)] = (
                s * MAX_LEN + iota + (l0 + j * CH + c * LANES))

    combs = []
    for j in range(NCH):
        toks[j].wait()
        combs.append(
            pltpu.async_copy(comb_hbm.at[comb_idx.at[j]],
                             acc.at[pl.ds(j * CH, CH)], gsems[j], add=True))
    outs = []
    for j in range(NCH):
        combs[j].wait()
        outs.append(
            pltpu.async_copy(acc.at[pl.ds(j * CH, CH)],
                             out_hbm.at[pl.ds(base + j * CH, CH)], osem))
    for h in outs:
        h.wait()


@jax.jit
def kernel(x, segment, token_table, segment_table, pe):
    comb = _comb_table(segment_table, pe).reshape(NSEG * MAX_LEN, HIDDEN)
    out = _embed_sc(token_table, comb, x, segment)
    return out.reshape(BATCH, MAX_LEN, HIDDEN)
